# Initial kernel scaffold; baseline (speedup 1.0000x reference)
import functools

import jax
import jax.numpy as jnp
from jax import lax
from jax.experimental import pallas as pl
from jax.experimental.pallas import tpu as pltpu
from jax.experimental.pallas import tpu_sc as plsc

D = 128
S = 16384
NC = 2
NS = 16
NW = NC * NS
C = 128
TOK_PER_W = S // NW
NCHUNK = TOK_PER_W // C
NT = 5


def _sc_body(table_hbm, idx_hbm, sidx_hbm, out_hbm,
             idx_v, sidx_v, acc_v, stage_v, gsem, ssem):
    wid = lax.axis_index("s") * NC + lax.axis_index("c")
    pltpu.sync_copy(sidx_hbm, sidx_v)
    for ci in range(NCHUNK):
        base = (wid * NCHUNK + ci) * C
        pltpu.sync_copy(idx_hbm.at[wid, ci], idx_v)
        cps = [pltpu.async_copy(table_hbm.at[idx_v.at[0]], acc_v, gsem)]
        for t in range(NT - 1):
            cps.append(pltpu.async_copy(table_hbm.at[idx_v.at[t + 1]],
                                        stage_v.at[t], gsem))
        for cp in cps:
            cp.wait()
        adds = [pltpu.async_copy(stage_v.at[t], acc_v.at[sidx_v.at[t]],
                                 ssem, add=True)
                for t in range(NT - 1)]
        for cp in adds:
            cp.wait()
        pltpu.sync_copy(acc_v, out_hbm.at[pl.ds(base, C)])


_sc_gather_sum = functools.partial(
    pl.kernel,
    out_type=jax.ShapeDtypeStruct((S, D), jnp.float32),
    mesh=plsc.VectorSubcoreMesh(core_axis_name="c", subcore_axis_name="s"),
    scratch_types=[
        pltpu.VMEM((NT, C), jnp.int32),
        pltpu.VMEM((NT - 1, C), jnp.int32),
        pltpu.VMEM((C, D), jnp.float32),
        pltpu.VMEM((NT - 1, C, D), jnp.float32),
        pltpu.SemaphoreType.DMA,
        pltpu.SemaphoreType.DMA,
    ],
)(_sc_body)


def _tc_body(v_ref, g_ref, w1_ref, b1_ref, w2_ref, b2_ref, gm_ref, bt_ref,
             o_ref):
    v = v_ref[...]
    x = jnp.sign(v) * jnp.log1p(jnp.abs(v))
    h1 = x * w1_ref[...] + b1_ref[...]
    h1 = 0.5 * h1 * (1.0 + lax.erf(h1 * 0.7071067811865475))
    h2 = (jnp.dot(h1, w2_ref[...], preferred_element_type=jnp.float32)
          + b2_ref[...] + g_ref[...])
    mean = jnp.mean(h2, axis=-1, keepdims=True)
    xc = h2 - mean
    var = jnp.mean(xc * xc, axis=-1, keepdims=True)
    o_ref[...] = xc * lax.rsqrt(var + 1e-5) * gm_ref[...] + bt_ref[...]


BT = 2048


def _tc_mlp_ln(v2, g, W1, b1, W2, b2, gamma, beta):
    row = pl.BlockSpec((1, D), lambda i: (0, 0))
    return pl.pallas_call(
        _tc_body,
        grid=(S // BT,),
        in_specs=[
            pl.BlockSpec((BT, 1), lambda i: (i, 0)),
            pl.BlockSpec((BT, D), lambda i: (i, 0)),
            row, row, pl.BlockSpec((D, D), lambda i: (0, 0)), row, row, row,
        ],
        out_specs=pl.BlockSpec((BT, D), lambda i: (i, 0)),
        out_shape=jax.ShapeDtypeStruct((S, D), jnp.float32),
    )(v2, g, W1, b1, W2, b2, gamma, beta)


def kernel(values, field_idx, family_idx, entity_type_idx, entity_id, dim_idx,
           field_emb, family_emb, entity_type_emb, entity_id_emb, dim_emb,
           W1, b1, W2, b2, gamma, beta):
    table = jnp.concatenate(
        [field_emb, family_emb, entity_type_emb, entity_id_emb, dim_emb],
        axis=0)
    idx = jnp.stack([
        field_idx,
        family_idx + 32,
        entity_type_idx + 40,
        entity_id + 44,
        dim_idx + 108,
    ], axis=0).astype(jnp.int32)
    idx = idx.reshape(NT, NW, NCHUNK, C).transpose(1, 2, 0, 3)
    sidx = jnp.broadcast_to(jnp.arange(C, dtype=jnp.int32), (NT - 1, C))
    g = _sc_gather_sum(table, idx, sidx)
    return _tc_mlp_ln(values.reshape(S, 1), g,
                      W1, b1.reshape(1, D), W2, b2.reshape(1, D),
                      gamma.reshape(1, D), beta.reshape(1, D))


# speedup vs baseline: 1.5186x; 1.5186x over previous
import functools

import jax
import jax.numpy as jnp
from jax import lax
from jax.experimental import pallas as pl
from jax.experimental.pallas import tpu as pltpu
from jax.experimental.pallas import tpu_sc as plsc

D = 128
S = 16384
NC = 2
NS = 16
NW = NC * NS
C = 128
TOK_PER_W = S // NW
NCHUNK = TOK_PER_W // C
NT = 5


def _sc_body(table_hbm, idx_hbm, sidx_hbm, out_hbm,
             idx_v, sidx_v, stage_v, acc_sh, gsem, ssem):
    cid = lax.axis_index("c")
    sid = lax.axis_index("s")
    wid = sid * NC + cid
    pltpu.sync_copy(sidx_hbm.at[sid], sidx_v)
    for ci in range(NCHUNK):
        base = (wid * NCHUNK + ci) * C
        pltpu.sync_copy(idx_hbm.at[wid, ci], idx_v)
        cps = [pltpu.async_copy(table_hbm.at[idx_v.at[t]], stage_v.at[t], gsem)
               for t in range(NT)]
        for cp in cps:
            cp.wait()
        pltpu.sync_copy(stage_v.at[0], acc_sh.at[pl.ds(sid * C, C)])
        adds = [pltpu.async_copy(stage_v.at[t + 1], acc_sh.at[sidx_v.at[t]],
                                 ssem, add=True)
                for t in range(NT - 1)]
        for cp in adds:
            cp.wait()
        pltpu.sync_copy(acc_sh.at[pl.ds(sid * C, C)],
                        out_hbm.at[pl.ds(base, C)])


@functools.cache
def _sc_gather_sum_fn():
    return pl.kernel(
        _sc_body,
        out_type=jax.ShapeDtypeStruct((S, D), jnp.float32),
        mesh=plsc.VectorSubcoreMesh(core_axis_name="c", subcore_axis_name="s",
                                    num_cores=NC, num_subcores=NS),
        scratch_types=[
            pltpu.VMEM((NT, C), jnp.int32),
            pltpu.VMEM((NT - 1, C), jnp.int32),
            pltpu.VMEM((NT, C, D), jnp.float32),
            pltpu.VMEM_SHARED((NS * C, D), jnp.float32),
            pltpu.SemaphoreType.DMA,
            pltpu.SemaphoreType.DMA,
        ],
    )


def _tc_body(v_ref, g_ref, w1_ref, b1_ref, w2_ref, b2_ref, gm_ref, bt_ref,
             o_ref):
    v = v_ref[...]
    x = jnp.sign(v) * jnp.log1p(jnp.abs(v))
    h1 = x * w1_ref[...] + b1_ref[...]
    h1 = 0.5 * h1 * (1.0 + lax.erf(h1 * 0.7071067811865475))
    h2 = (jnp.dot(h1, w2_ref[...], preferred_element_type=jnp.float32)
          + b2_ref[...] + g_ref[...])
    mean = jnp.mean(h2, axis=-1, keepdims=True)
    xc = h2 - mean
    var = jnp.mean(xc * xc, axis=-1, keepdims=True)
    o_ref[...] = xc * lax.rsqrt(var + 1e-5) * gm_ref[...] + bt_ref[...]


BT = 2048


def _tc_mlp_ln(v2, g, W1, b1, W2, b2, gamma, beta):
    row = pl.BlockSpec((1, D), lambda i: (0, 0))
    return pl.pallas_call(
        _tc_body,
        grid=(S // BT,),
        in_specs=[
            pl.BlockSpec((BT, 1), lambda i: (i, 0)),
            pl.BlockSpec((BT, D), lambda i: (i, 0)),
            row, row, pl.BlockSpec((D, D), lambda i: (0, 0)), row, row, row,
        ],
        out_specs=pl.BlockSpec((BT, D), lambda i: (i, 0)),
        out_shape=jax.ShapeDtypeStruct((S, D), jnp.float32),
    )(v2, g, W1, b1, W2, b2, gamma, beta)


def kernel(values, field_idx, family_idx, entity_type_idx, entity_id, dim_idx,
           field_emb, family_emb, entity_type_emb, entity_id_emb, dim_emb,
           W1, b1, W2, b2, gamma, beta):
    table = jnp.concatenate(
        [field_emb, family_emb, entity_type_emb, entity_id_emb, dim_emb],
        axis=0)
    idx = jnp.stack([
        field_idx,
        family_idx + 32,
        entity_type_idx + 40,
        entity_id + 44,
        dim_idx + 108,
    ], axis=0).astype(jnp.int32)
    idx = idx.reshape(NT, NW, NCHUNK, C).transpose(1, 2, 0, 3)
    sidx = (jnp.arange(NS, dtype=jnp.int32)[:, None, None] * C
            + jnp.broadcast_to(jnp.arange(C, dtype=jnp.int32), (NT - 1, C)))
    g = _sc_gather_sum_fn()(table, idx, sidx)
    return _tc_mlp_ln(values.reshape(S, 1), g,
                      W1, b1.reshape(1, D), W2, b2.reshape(1, D),
                      gamma.reshape(1, D), beta.reshape(1, D))


# speedup vs baseline: 4.6002x; 3.0293x over previous
import functools

import jax
import jax.numpy as jnp
from jax import lax
from jax.experimental import pallas as pl
from jax.experimental.pallas import tpu as pltpu
from jax.experimental.pallas import tpu_sc as plsc

D = 128
S = 16384
NC = 2
NS = 16
NW = NC * NS
C = 128
TOK_PER_W = S // NW
NCHUNK = TOK_PER_W // C
NT = 3
NBUF = 2
L = 16
EID_OFF = 1024
DIM_OFF = 1088


def _sc_body(table_hbm, fld_hbm, fam_hbm, ety_hbm, eid_hbm, dim_hbm, out_hbm,
             raw_v, idx_v, stage_v, gsem, osem):
    cid = lax.axis_index("c")
    sid = lax.axis_index("s")
    wid = sid * NC + cid
    wbase = wid * TOK_PER_W
    for t, ref in enumerate((fld_hbm, fam_hbm, ety_hbm, eid_hbm, dim_hbm)):
        pltpu.sync_copy(ref.at[pl.ds(wbase, TOK_PER_W)], raw_v.at[t])

    def build_idx(ci):
        b = ci % NBUF
        for g in range(C // L):
            sl = pl.ds(ci * C + g * L, L)
            dsl = pl.ds(g * L, L)
            ffe = (raw_v[0, sl] * 32 + raw_v[1, sl] * 4) + raw_v[2, sl]
            idx_v[b, 0, dsl] = ffe
            idx_v[b, 1, dsl] = raw_v[3, sl] + EID_OFF
            idx_v[b, 2, dsl] = raw_v[4, sl] + DIM_OFF

    def issue_gathers(ci):
        b = ci % NBUF
        return [pltpu.async_copy(table_hbm.at[idx_v.at[b, t]],
                                 stage_v.at[b, t], gsem)
                for t in range(NT)]

    def accumulate(ci):
        b = ci % NBUF

        def row_body(r, carry):
            for g in range(D // L):
                sl = pl.ds(g * L, L)
                stage_v[b, 0, r, sl] = (stage_v[b, 0, r, sl]
                                        + stage_v[b, 1, r, sl]
                                        + stage_v[b, 2, r, sl])
            return carry

        lax.fori_loop(0, C, row_body, 0)

    gh = {}
    oh = {}
    for ci in range(NCHUNK + 1):
        if ci < NCHUNK:
            if ci - NBUF in oh:
                oh.pop(ci - NBUF).wait()
            build_idx(ci)
            gh[ci] = issue_gathers(ci)
        if ci >= 1:
            cj = ci - 1
            for h in gh.pop(cj):
                h.wait()
            accumulate(cj)
            oh[cj] = pltpu.async_copy(
                stage_v.at[cj % NBUF, 0],
                out_hbm.at[pl.ds(wbase + cj * C, C)], osem)
    for h in oh.values():
        h.wait()


@functools.cache
def _sc_gather_sum_fn():
    return pl.kernel(
        _sc_body,
        out_type=jax.ShapeDtypeStruct((S, D), jnp.float32),
        mesh=plsc.VectorSubcoreMesh(core_axis_name="c", subcore_axis_name="s",
                                    num_cores=NC, num_subcores=NS),
        compiler_params=pltpu.CompilerParams(use_tc_tiling_on_sc=False),
        scratch_types=[
            pltpu.VMEM((5, TOK_PER_W), jnp.int32),
            pltpu.VMEM((NBUF, NT, C), jnp.int32),
            pltpu.VMEM((NBUF, NT, C, D), jnp.float32),
            pltpu.SemaphoreType.DMA,
            pltpu.SemaphoreType.DMA,
        ],
    )


def _tc_body(v_ref, g_ref, w1_ref, b1_ref, w2_ref, b2_ref, gm_ref, bt_ref,
             o_ref):
    v = v_ref[...]
    x = jnp.sign(v) * jnp.log1p(jnp.abs(v))
    h1 = x * w1_ref[...] + b1_ref[...]
    h1 = 0.5 * h1 * (1.0 + lax.erf(h1 * 0.7071067811865475))
    h2 = (jnp.dot(h1, w2_ref[...], preferred_element_type=jnp.float32)
          + b2_ref[...] + g_ref[...])
    mean = jnp.mean(h2, axis=-1, keepdims=True)
    xc = h2 - mean
    var = jnp.mean(xc * xc, axis=-1, keepdims=True)
    o_ref[...] = xc * lax.rsqrt(var + 1e-5) * gm_ref[...] + bt_ref[...]


BT = 2048


def _tc_mlp_ln(v2, g, W1, b1, W2, b2, gamma, beta):
    row = pl.BlockSpec((1, D), lambda i: (0, 0))
    return pl.pallas_call(
        _tc_body,
        grid=(S // BT,),
        in_specs=[
            pl.BlockSpec((BT, 1), lambda i: (i, 0)),
            pl.BlockSpec((BT, D), lambda i: (i, 0)),
            row, row, pl.BlockSpec((D, D), lambda i: (0, 0)), row, row, row,
        ],
        out_specs=pl.BlockSpec((BT, D), lambda i: (i, 0)),
        out_shape=jax.ShapeDtypeStruct((S, D), jnp.float32),
    )(v2, g, W1, b1, W2, b2, gamma, beta)


def kernel(values, field_idx, family_idx, entity_type_idx, entity_id, dim_idx,
           field_emb, family_emb, entity_type_emb, entity_id_emb, dim_emb,
           W1, b1, W2, b2, gamma, beta):
    ffe = (field_emb[:, None, None, :] + family_emb[None, :, None, :]
           + entity_type_emb[None, None, :, :]).reshape(32 * 8 * 4, D)
    table = jnp.concatenate([ffe, entity_id_emb, dim_emb], axis=0)
    g = _sc_gather_sum_fn()(table, field_idx, family_idx, entity_type_idx,
                            entity_id, dim_idx)
    return _tc_mlp_ln(values.reshape(S, 1), g,
                      W1, b1.reshape(1, D), W2, b2.reshape(1, D),
                      gamma.reshape(1, D), beta.reshape(1, D))
